# Initial kernel scaffold; baseline (speedup 1.0000x reference)
import jax
import jax.numpy as jnp
from jax.experimental import pallas as pl


def kernel(image_inputs, text_inputs, targets, features):
    raise NotImplementedError("write your pallas kernel here")


# speedup vs baseline: 5.5624x; 5.5624x over previous
import functools

import jax
import jax.numpy as jnp
from jax import lax
from jax.experimental import pallas as pl
from jax.experimental.pallas import tpu as pltpu
from jax.experimental.pallas import tpu_sc as plsc

NUM_SAMPLES = 100000
NUM_FEATURES = 128
BATCH = 1024
TEMP = 0.02
LOGIT_BOUND = 1.0 / TEMP

BLOCK_N = 2000
NB = NUM_SAMPLES // BLOCK_N


def _tc_body(img_ref, feat_ref, g_ref, out_ref, img_scr, acc_scr):
    step = pl.program_id(0)

    @pl.when(step == 0)
    def _init():
        img = img_ref[...]
        n = jnp.sqrt(jnp.sum(img * img, axis=1, keepdims=True))
        img_scr[...] = img / (jnp.maximum(n, 1e-12) * TEMP)
        acc_scr[...] = jnp.zeros_like(acc_scr)

    logits = lax.dot_general(
        img_scr[...], feat_ref[...], (((1,), (1,)), ((), ())),
        preferred_element_type=jnp.float32)
    acc_scr[...] += jnp.sum(jnp.exp(logits - LOGIT_BOUND), axis=1, keepdims=True)

    @pl.when(step == NB - 1)
    def _fin():
        tgt = jnp.sum(img_scr[...] * g_ref[...], axis=1, keepdims=True)
        lse = LOGIT_BOUND + jnp.log(acc_scr[...])
        out_ref[...] = jnp.mean(lse - tgt).reshape(1, 1)


def _tc_loss(img, feats, g):
    out = pl.pallas_call(
        _tc_body,
        grid=(NB,),
        in_specs=[
            pl.BlockSpec((BATCH, NUM_FEATURES), lambda i: (0, 0)),
            pl.BlockSpec((BLOCK_N, NUM_FEATURES), lambda i: (i, 0)),
            pl.BlockSpec((BATCH, NUM_FEATURES), lambda i: (0, 0)),
        ],
        out_specs=pl.BlockSpec((1, 1), lambda i: (0, 0)),
        out_shape=jax.ShapeDtypeStruct((1, 1), jnp.float32),
        scratch_shapes=[
            pltpu.VMEM((BATCH, NUM_FEATURES), jnp.float32),
            pltpu.VMEM((BATCH, 1), jnp.float32),
        ],
        compiler_params=pltpu.CompilerParams(
            dimension_semantics=("arbitrary",)),
    )(img, feats, g)
    return out[0, 0]


def _make_sc_gather():
    info = plsc.get_sparse_core_info()
    nc, ns = info.num_cores, info.num_subcores
    nw = nc * ns
    b_per_w = BATCH // nw
    mesh = plsc.VectorSubcoreMesh(core_axis_name="c", subcore_axis_name="s")

    @functools.partial(
        pl.kernel, mesh=mesh,
        out_type=jax.ShapeDtypeStruct((BATCH, NUM_FEATURES), jnp.float32),
        scratch_types=[
            pltpu.VMEM((b_per_w,), jnp.int32),
            pltpu.VMEM((b_per_w, NUM_FEATURES), jnp.float32),
            pltpu.SemaphoreType.DMA,
        ],
    )
    def sc_gather(table_hbm, idx_hbm, out_hbm, idx_v, rows_v, sem):
        wid = lax.axis_index("s") * nc + lax.axis_index("c")
        base = wid * b_per_w
        pltpu.sync_copy(idx_hbm.at[pl.ds(base, b_per_w)], idx_v)
        pltpu.async_copy(table_hbm.at[idx_v], rows_v, sem).wait()
        pltpu.sync_copy(rows_v, out_hbm.at[pl.ds(base, b_per_w)])

    return sc_gather


def kernel(image_inputs, text_inputs, targets, features):
    del text_inputs
    g = _make_sc_gather()(features, targets.astype(jnp.int32))
    return _tc_loss(image_inputs, features, g)


# speedup vs baseline: 5.6729x; 1.0199x over previous
import functools
import math

import jax
import jax.numpy as jnp
from jax import lax
from jax.experimental import pallas as pl
from jax.experimental.pallas import tpu as pltpu
from jax.experimental.pallas import tpu_sc as plsc

NUM_SAMPLES = 100000
NUM_FEATURES = 128
BATCH = 1024
TEMP = 0.02
LOG2E = math.log2(math.e)
BOUND2 = 50.0 * LOG2E

BLOCK_N = 2000
NB = NUM_SAMPLES // BLOCK_N


def _prep_body(img_ref, img2_ref):
    img = img_ref[...]
    n = jnp.sqrt(jnp.sum(img * img, axis=1, keepdims=True))
    img2_ref[...] = img * (LOG2E / TEMP / jnp.maximum(n, 1e-12))


def _stream_body(img2_ref, feat_ref, acc_ref, acc_scr):
    step = pl.program_id(0)

    @pl.when(step == 0)
    def _init():
        acc_scr[...] = jnp.zeros_like(acc_scr)

    z = lax.dot_general(
        img2_ref[...], feat_ref[...], (((1,), (1,)), ((), ())),
        preferred_element_type=jnp.float32)
    acc_scr[...] += jnp.sum(jnp.exp2(z - BOUND2), axis=1, keepdims=True)

    @pl.when(step == NB - 1)
    def _fin():
        acc_ref[...] = acc_scr[...]


def _combine_body(img2_ref, g_ref, acc_ref, out_ref):
    tgt = jnp.sum(img2_ref[...] * g_ref[...], axis=1, keepdims=True) * (1.0 / LOG2E)
    lse = 50.0 + jnp.log(acc_ref[...])
    out_ref[...] = jnp.mean(lse - tgt).reshape(1, 1)


def _tc_loss(img, feats, g):
    img2 = pl.pallas_call(
        _prep_body,
        out_shape=jax.ShapeDtypeStruct((BATCH, NUM_FEATURES), jnp.float32),
    )(img)
    acc = pl.pallas_call(
        _stream_body,
        grid=(NB,),
        in_specs=[
            pl.BlockSpec((BATCH, NUM_FEATURES), lambda i: (0, 0)),
            pl.BlockSpec((BLOCK_N, NUM_FEATURES), lambda i: (i, 0)),
        ],
        out_specs=pl.BlockSpec((BATCH, 1), lambda i: (0, 0)),
        out_shape=jax.ShapeDtypeStruct((BATCH, 1), jnp.float32),
        scratch_shapes=[pltpu.VMEM((BATCH, 1), jnp.float32)],
        compiler_params=pltpu.CompilerParams(
            dimension_semantics=("arbitrary",)),
    )(img2, feats)
    out = pl.pallas_call(
        _combine_body,
        out_shape=jax.ShapeDtypeStruct((1, 1), jnp.float32),
    )(img2, g, acc)
    return out[0, 0]


def _make_sc_gather():
    info = plsc.get_sparse_core_info()
    nc, ns = info.num_cores, info.num_subcores
    nw = nc * ns
    b_per_w = BATCH // nw
    mesh = plsc.VectorSubcoreMesh(core_axis_name="c", subcore_axis_name="s")

    @functools.partial(
        pl.kernel, mesh=mesh,
        out_type=jax.ShapeDtypeStruct((BATCH, NUM_FEATURES), jnp.float32),
        scratch_types=[
            pltpu.VMEM((b_per_w,), jnp.int32),
            pltpu.VMEM((b_per_w, NUM_FEATURES), jnp.float32),
            pltpu.SemaphoreType.DMA,
        ],
    )
    def sc_gather(table_hbm, idx_hbm, out_hbm, idx_v, rows_v, sem):
        wid = lax.axis_index("s") * nc + lax.axis_index("c")
        base = wid * b_per_w
        pltpu.sync_copy(idx_hbm.at[pl.ds(base, b_per_w)], idx_v)
        pltpu.async_copy(table_hbm.at[idx_v], rows_v, sem).wait()
        pltpu.sync_copy(rows_v, out_hbm.at[pl.ds(base, b_per_w)])

    return sc_gather


def kernel(image_inputs, text_inputs, targets, features):
    del text_inputs
    g = _make_sc_gather()(features, targets.astype(jnp.int32))
    return _tc_loss(image_inputs, features, g)
